# baseline (speedup 1.0000x reference)
import functools

import numpy as np
import jax
import jax.numpy as jnp
from jax import lax
from jax.experimental import pallas as pl
from jax.experimental.pallas import tpu as pltpu
from jax.experimental.pallas import tpu_sc as plsc

VOCAB = 1_000_000
DIM = 64
BATCH = 16384
NEG = 20
NCTX = NEG + 1
NW = 32
BPW = BATCH // NW
CH = 32
NCHUNK = BPW // CH
NIP = 8
SCORES = NCTX * CH
NSLICE = tuple((128 * j, 128) for j in range(5))

O_PERM = [0, 8, 4, 12, 2, 10, 6, 14, 1, 9, 5, 13, 3, 11, 7, 15]


TW = 2048
TGRID = -(-VOCAB // (2 * TW))
VPAD = TGRID * 2 * TW


def _tr_body(alo_ref, ahi_ref, blo_ref, bhi_ref, ao_ref, bo_ref):
  ao_ref[...] = jnp.concatenate(
      [alo_ref[...].T, ahi_ref[...].T], axis=1).astype(jnp.bfloat16)
  bo_ref[...] = jnp.concatenate(
      [blo_ref[...].T, bhi_ref[...].T], axis=1).astype(jnp.bfloat16)


def _transpose_tables(center_table, context_table):
  a = center_table.T
  b = context_table.T
  nin = -(-VOCAB // TW) - 1
  lo = pl.BlockSpec((DIM, TW), lambda w: (0, jnp.minimum(2 * w, nin)))
  hi = pl.BlockSpec((DIM, TW), lambda w: (0, jnp.minimum(2 * w + 1, nin)))
  out = pl.BlockSpec((TW, 2 * DIM), lambda w: (w, 0))
  return pl.pallas_call(
      _tr_body,
      grid=(TGRID,),
      in_specs=[lo, hi, lo, hi],
      out_specs=[out, out],
      out_shape=[jax.ShapeDtypeStruct((VPAD // 2, 2 * DIM), jnp.bfloat16)] * 2,
  )(a, a, b, b)


def _sc_scores(center_words, context_words, neg2d, center_table, context_table):
  mesh = plsc.VectorSubcoreMesh(core_axis_name="c", subcore_axis_name="s")

  scratch = []
  for _ in range(2):
    scratch += [
        pltpu.VMEM((CH,), jnp.int32),
        pltpu.VMEM((CH,), jnp.int32),
        pltpu.VMEM((NIP, 128), jnp.int32),
        pltpu.VMEM((CH,), jnp.int32),
        pltpu.VMEM((CH,), jnp.int32),
        pltpu.VMEM((NIP, 128), jnp.int32),
        pltpu.VMEM((CH, 2 * DIM), jnp.bfloat16),
        pltpu.VMEM((CH, 2 * DIM), jnp.bfloat16),
        pltpu.VMEM((CH * NEG, 2 * DIM), jnp.bfloat16),
        pltpu.VMEM((SCORES,), jnp.float32),
        pltpu.SemaphoreType.DMA,
    ]

  @functools.partial(
      pl.kernel,
      mesh=mesh,
      compiler_params=pltpu.CompilerParams(
          needs_layout_passes=False, use_tc_tiling_on_sc=False),
      out_type=jax.ShapeDtypeStruct((BATCH * NCTX,), jnp.float32),
      scratch_types=scratch,
  )
  def k(cw_hbm, xw_hbm, nw_hbm, ctab2_hbm, xtab2_hbm, out_hbm, *bufs):
    wid = lax.axis_index("s") * 2 + lax.axis_index("c")
    sets = (bufs[:11], bufs[11:])

    iota = lax.iota(jnp.int32, 16)
    rots, conds = {}, {}
    for g in (16, 8, 4, 2):
      rots[g] = (iota & ~(g - 1)) | ((iota + g // 2) & (g - 1))
      conds[g] = (iota & (g - 1)) < (g // 2)

    def _merge(a, b, g):
      asum = None if a is None else a + a.at[rots[g]].get(mode="promise_in_bounds")
      bsum = None if b is None else b + b.at[rots[g]].get(mode="promise_in_bounds")
      if asum is None and bsum is None:
        return None
      if bsum is None:
        return jnp.where(conds[g], asum, 0.0)
      if asum is None:
        return jnp.where(conds[g], 0.0, bsum)
      return jnp.where(conds[g], asum, bsum)

    def _tree(leaves):
      regs = list(leaves)
      g = 16
      while len(regs) > 1:
        regs = [_merge(regs[2 * i], regs[2 * i + 1], g)
                for i in range(len(regs) // 2)]
        g //= 2
      return regs[0]

    brev = (((iota & 1) * 8) | ((iota & 2) * 2)
            | ((iota & 4) // 2) | ((iota & 8) // 8))
    ovec1 = brev * CH
    mask2 = brev < 5
    ovec2 = jnp.where(mask2, (16 + brev) * CH, 0)

    def _nidx_slice(nu, off, cnt):
      if cnt == 128:
        return nu.at[off // 128]
      return nu.at[off // 128, pl.ds(0, cnt)]

    def fire(s, ci):
      cidx, xidx, nidx, cu, xu, nu, crows, xrows, nrows, _, sem = sets[s]
      base = pl.multiple_of(wid * BPW + ci * CH, CH)
      nbase = pl.multiple_of((wid * NCHUNK + ci) * NIP, NIP)
      pltpu.sync_copy(cw_hbm.at[pl.ds(base, CH)], cidx)
      pltpu.sync_copy(xw_hbm.at[pl.ds(base, CH)], xidx)
      pltpu.sync_copy(nw_hbm.at[pl.ds(nbase, NIP)], nidx)
      for q in range(CH // 16):
        cu[pl.ds(16 * q, 16)] = cidx[pl.ds(16 * q, 16)] // 2
        xu[pl.ds(16 * q, 16)] = xidx[pl.ds(16 * q, 16)] // 2
      for j in range(5):
        for q in range(8):
          nu[j, pl.ds(16 * q, 16)] = nidx[j, pl.ds(16 * q, 16)] // 2
      pltpu.async_copy(ctab2_hbm.at[cu], crows, sem)
      pltpu.async_copy(xtab2_hbm.at[xu], xrows, sem)
      for off, cnt in NSLICE:
        pltpu.async_copy(xtab2_hbm.at[_nidx_slice(nu, off, cnt)],
                         nrows.at[pl.ds(off, cnt)], sem)

    def drain(s):
      cidx, xidx, nidx, cu, xu, nu, crows, xrows, nrows, _, sem = sets[s]
      pltpu.make_async_copy(ctab2_hbm.at[cu], crows, sem).wait()
      pltpu.make_async_copy(xtab2_hbm.at[xu], xrows, sem).wait()
      for off, cnt in NSLICE:
        pltpu.make_async_copy(xtab2_hbm.at[_nidx_slice(nu, off, cnt)],
                              nrows.at[pl.ds(off, cnt)], sem).wait()

    def _pick(vec, lane):
      return vec.at[jnp.full((16,), lane, jnp.int32)].get(
          mode="promise_in_bounds")[0]

    def _half(ref, t, off):
      out = []
      for q in range(2):
        a, b = plsc.unpack(ref[t, pl.ds(off + 32 * q, 32)],
                           format=plsc.PackFormat.INTERLEAVED)
        out += [a, b]
      return out

    def compute(s, ci):
      cidx, xidx, nidx, _, _, _, crows, xrows, nrows, scores, _ = sets[s]

      def row(r, carry):
        cb = (_pick(cidx[pl.ds(r // 16 * 16, 16)], r % 16) & 1) * 64
        xb = (_pick(xidx[pl.ds(r // 16 * 16, 16)], r % 16) & 1) * 64
        c = _half(crows, r, cb)
        x = _half(xrows, r, xb)
        ps = [c[0] * x[0] + c[1] * x[1] + c[2] * x[2] + c[3] * x[3]]
        for n in range(NEG):
          t = r * NEG + n
          nvec = nidx[t // 128, pl.ds((t % 128) // 16 * 16, 16)]
          nb = (_pick(nvec, t % 16) & 1) * 64
          v = _half(nrows, t, nb)
          ps.append(c[0] * v[0] + c[1] * v[1] + c[2] * v[2] + c[3] * v[3])
        f1 = _tree(ps[:16])
        f2 = _tree(ps[16:] + [None] * 11)
        plsc.store_scatter(scores, [ovec1 + r], f1)
        plsc.store_scatter(scores, [ovec2 + r], f2, mask=mask2)
        return carry

      lax.fori_loop(0, CH, row, 0)
      obase = pl.multiple_of((wid * NCHUNK + ci) * SCORES, 8)
      pltpu.sync_copy(scores, out_hbm.at[pl.ds(obase, SCORES)])

    fire(0, 0)

    def pair(ip, carry):
      ci = ip * 2
      fire(1, ci + 1)
      drain(0)
      compute(0, ci)

      @pl.when(ip < NCHUNK // 2 - 1)
      def _():
        fire(0, ci + 2)

      drain(1)
      compute(1, ci + 1)
      return carry

    lax.fori_loop(0, NCHUNK // 2, pair, 0)

  return k(center_words, context_words, neg2d, center_table, context_table)


def _loss_body(s_ref, o_ref):
  x = s_ref[...]
  flat = (lax.broadcasted_iota(jnp.int32, x.shape, 0) * 128
          + lax.broadcasted_iota(jnp.int32, x.shape, 1))
  n_id = (flat % SCORES) // CH
  t = jnp.where(n_id == 0, x, -x)
  p = 1.0 / (1.0 + jnp.exp(-t))
  ll = -jnp.log(p + 1e-10)
  o_ref[0, 0] = jnp.sum(ll) * (1.0 / BATCH)


def kernel(center_words, context_words, negative_samples, center_table, context_table):
  ct2, xt2 = _transpose_tables(center_table, context_table)

  def remap(v):
    v = v.astype(jnp.int32)
    p = v % 4096
    return 4096 * (v // 4096) + 2 * (p % 2048) + p // 2048

  cw = remap(center_words)
  xw = remap(context_words)
  ns = remap(negative_samples).reshape(BATCH // CH, CH * NEG)
  ns = jnp.pad(ns, ((0, 0), (0, NIP * 128 - CH * NEG)))
  ns = ns.reshape(BATCH // CH * NIP, 128)
  scores = _sc_scores(cw, xw, ns, ct2, xt2)
  s2 = scores.reshape(BATCH * NCTX // 128, 128)
  loss = pl.pallas_call(
      _loss_body,
      out_shape=jax.ShapeDtypeStruct((1, 1), jnp.float32),
      out_specs=pl.BlockSpec(memory_space=pltpu.SMEM),
  )(s2)
  return loss[0, 0]


# speedup vs baseline: 2.2312x; 2.2312x over previous
import functools

import numpy as np
import jax
import jax.numpy as jnp
from jax import lax
from jax.experimental import pallas as pl
from jax.experimental.pallas import tpu as pltpu
from jax.experimental.pallas import tpu_sc as plsc

VOCAB = 1_000_000
DIM = 64
BATCH = 16384
NEG = 20
NCTX = NEG + 1
NW = 32
BPW = BATCH // NW
CH = 16
NCHUNK = BPW // CH
NIP = 8
SCORES = NCTX * CH
NSLICE = ((0, 128), (128, 128), (256, 64))

O_PERM = [0, 8, 4, 12, 2, 10, 6, 14, 1, 9, 5, 13, 3, 11, 7, 15]


TW = 4096
TGRID = -(-VOCAB // (2 * TW))
VPAD = TGRID * 2 * TW


def _tr_body(alo_ref, ahi_ref, blo_ref, bhi_ref, ao_ref, bo_ref):
  ao_ref[...] = jnp.concatenate([alo_ref[...].T, ahi_ref[...].T], axis=1)
  bo_ref[...] = jnp.concatenate([blo_ref[...].T, bhi_ref[...].T], axis=1)


def _transpose_tables(center_table, context_table):
  a = center_table.T
  b = context_table.T
  nin = -(-VOCAB // TW) - 1
  lo = pl.BlockSpec((DIM, TW), lambda w: (0, jnp.minimum(2 * w, nin)))
  hi = pl.BlockSpec((DIM, TW), lambda w: (0, jnp.minimum(2 * w + 1, nin)))
  out = pl.BlockSpec((TW, 2 * DIM), lambda w: (w, 0))
  return pl.pallas_call(
      _tr_body,
      grid=(TGRID,),
      in_specs=[lo, hi, lo, hi],
      out_specs=[out, out],
      out_shape=[jax.ShapeDtypeStruct((VPAD // 2, 2 * DIM), jnp.float32)] * 2,
  )(a, a, b, b)


def _sc_scores(center_words, context_words, neg2d, center_table, context_table):
  mesh = plsc.VectorSubcoreMesh(core_axis_name="c", subcore_axis_name="s")

  scratch = []
  for _ in range(2):
    scratch += [
        pltpu.VMEM((CH,), jnp.int32),
        pltpu.VMEM((CH,), jnp.int32),
        pltpu.VMEM((NIP, 128), jnp.int32),
        pltpu.VMEM((CH,), jnp.int32),
        pltpu.VMEM((CH,), jnp.int32),
        pltpu.VMEM((NIP, 128), jnp.int32),
        pltpu.VMEM((CH, 2 * DIM), jnp.float32),
        pltpu.VMEM((CH, 2 * DIM), jnp.float32),
        pltpu.VMEM((CH * NEG, 2 * DIM), jnp.float32),
        pltpu.VMEM((SCORES,), jnp.float32),
        pltpu.SemaphoreType.DMA,
    ]

  @functools.partial(
      pl.kernel,
      mesh=mesh,
      compiler_params=pltpu.CompilerParams(
          needs_layout_passes=False, use_tc_tiling_on_sc=False),
      out_type=jax.ShapeDtypeStruct((BATCH * NCTX,), jnp.float32),
      scratch_types=scratch,
  )
  def k(cw_hbm, xw_hbm, nw_hbm, ctab2_hbm, xtab2_hbm, out_hbm, *bufs):
    wid = lax.axis_index("s") * 2 + lax.axis_index("c")
    sets = (bufs[:11], bufs[11:])

    iota = lax.iota(jnp.int32, 16)
    rots, conds = {}, {}
    for g in (16, 8, 4, 2):
      rots[g] = (iota & ~(g - 1)) | ((iota + g // 2) & (g - 1))
      conds[g] = (iota & (g - 1)) < (g // 2)

    def _merge(a, b, g):
      asum = None if a is None else a + a.at[rots[g]].get(mode="promise_in_bounds")
      bsum = None if b is None else b + b.at[rots[g]].get(mode="promise_in_bounds")
      if asum is None and bsum is None:
        return None
      if bsum is None:
        return jnp.where(conds[g], asum, 0.0)
      if asum is None:
        return jnp.where(conds[g], 0.0, bsum)
      return jnp.where(conds[g], asum, bsum)

    def _tree(leaves):
      regs = list(leaves)
      g = 16
      while len(regs) > 1:
        regs = [_merge(regs[2 * i], regs[2 * i + 1], g)
                for i in range(len(regs) // 2)]
        g //= 2
      return regs[0]

    brev = (((iota & 1) * 8) | ((iota & 2) * 2)
            | ((iota & 4) // 2) | ((iota & 8) // 8))
    ovec1 = brev * CH
    mask2 = brev < 5
    ovec2 = jnp.where(mask2, (16 + brev) * CH, 0)

    def _nidx_slice(nu, off, cnt):
      if cnt == 128:
        return nu.at[off // 128]
      return nu.at[off // 128, pl.ds(0, cnt)]

    def fire(s, ci):
      cidx, xidx, nidx, cu, xu, nu, crows, xrows, nrows, _, sem = sets[s]
      base = pl.multiple_of(wid * BPW + ci * CH, CH)
      nbase = pl.multiple_of((wid * NCHUNK + ci) * NIP, NIP)
      pltpu.sync_copy(cw_hbm.at[pl.ds(base, CH)], cidx)
      pltpu.sync_copy(xw_hbm.at[pl.ds(base, CH)], xidx)
      pltpu.sync_copy(nw_hbm.at[pl.ds(nbase, NIP)], nidx)
      for q in range(CH // 16):
        cu[pl.ds(16 * q, 16)] = cidx[pl.ds(16 * q, 16)] // 2
        xu[pl.ds(16 * q, 16)] = xidx[pl.ds(16 * q, 16)] // 2
      for j in range(3):
        for q in range(8):
          nu[j, pl.ds(16 * q, 16)] = nidx[j, pl.ds(16 * q, 16)] // 2
      pltpu.async_copy(ctab2_hbm.at[cu], crows, sem)
      pltpu.async_copy(xtab2_hbm.at[xu], xrows, sem)
      for off, cnt in NSLICE:
        pltpu.async_copy(xtab2_hbm.at[_nidx_slice(nu, off, cnt)],
                         nrows.at[pl.ds(off, cnt)], sem)

    def drain(s):
      cidx, xidx, nidx, cu, xu, nu, crows, xrows, nrows, _, sem = sets[s]
      pltpu.make_async_copy(ctab2_hbm.at[cu], crows, sem).wait()
      pltpu.make_async_copy(xtab2_hbm.at[xu], xrows, sem).wait()
      for off, cnt in NSLICE:
        pltpu.make_async_copy(xtab2_hbm.at[_nidx_slice(nu, off, cnt)],
                              nrows.at[pl.ds(off, cnt)], sem).wait()

    def _pick(vec, lane):
      return vec.at[jnp.full((16,), lane, jnp.int32)].get(
          mode="promise_in_bounds")[0]

    def _half(ref, t, off):
      return [ref[t, pl.ds(off + 16 * q, 16)] for q in range(4)]

    def compute(s, ci):
      cidx, xidx, nidx, _, _, _, crows, xrows, nrows, scores, _ = sets[s]

      def row(r, carry):
        cb = (_pick(cidx[pl.ds(r // 16 * 16, 16)], r % 16) & 1) * 64
        xb = (_pick(xidx[pl.ds(r // 16 * 16, 16)], r % 16) & 1) * 64
        c = _half(crows, r, cb)
        x = _half(xrows, r, xb)
        ps = [c[0] * x[0] + c[1] * x[1] + c[2] * x[2] + c[3] * x[3]]
        for n in range(NEG):
          t = r * NEG + n
          nvec = nidx[t // 128, pl.ds((t % 128) // 16 * 16, 16)]
          nb = (_pick(nvec, t % 16) & 1) * 64
          v = _half(nrows, t, nb)
          ps.append(c[0] * v[0] + c[1] * v[1] + c[2] * v[2] + c[3] * v[3])
        f1 = _tree(ps[:16])
        f2 = _tree(ps[16:] + [None] * 11)
        plsc.store_scatter(scores, [ovec1 + r], f1)
        plsc.store_scatter(scores, [ovec2 + r], f2, mask=mask2)
        return carry

      lax.fori_loop(0, CH, row, 0)
      obase = pl.multiple_of((wid * NCHUNK + ci) * SCORES, 8)
      pltpu.sync_copy(scores, out_hbm.at[pl.ds(obase, SCORES)])

    fire(0, 0)

    def pair(ip, carry):
      ci = ip * 2
      fire(1, ci + 1)
      drain(0)
      compute(0, ci)

      @pl.when(ip < NCHUNK // 2 - 1)
      def _():
        fire(0, ci + 2)

      drain(1)
      compute(1, ci + 1)
      return carry

    lax.fori_loop(0, NCHUNK // 2, pair, 0)

  return k(center_words, context_words, neg2d, center_table, context_table)


def _loss_body(s_ref, o_ref):
  x = s_ref[...]
  flat = (lax.broadcasted_iota(jnp.int32, x.shape, 0) * 128
          + lax.broadcasted_iota(jnp.int32, x.shape, 1))
  n_id = (flat % SCORES) // CH
  t = jnp.where(n_id == 0, x, -x)
  p = 1.0 / (1.0 + jnp.exp(-t))
  ll = -jnp.log(p + 1e-10)
  o_ref[0, 0] = jnp.sum(ll) * (1.0 / BATCH)


def kernel(center_words, context_words, negative_samples, center_table, context_table):
  ct2, xt2 = _transpose_tables(center_table, context_table)

  def remap(v):
    v = v.astype(jnp.int32)
    p = v % (2 * TW)
    return 2 * TW * (v // (2 * TW)) + 2 * (p % TW) + p // TW

  cw = remap(center_words)
  xw = remap(context_words)
  ns = remap(negative_samples).reshape(BATCH // CH, CH * NEG)
  ns = jnp.pad(ns, ((0, 0), (0, NIP * 128 - CH * NEG)))
  ns = ns.reshape(BATCH // CH * NIP, 128)
  scores = _sc_scores(cw, xw, ns, ct2, xt2)
  s2 = scores.reshape(BATCH * NCTX // 128, 128)
  loss = pl.pallas_call(
      _loss_body,
      out_shape=jax.ShapeDtypeStruct((1, 1), jnp.float32),
      out_specs=pl.BlockSpec(memory_space=pltpu.SMEM),
  )(s2)
  return loss[0, 0]


# speedup vs baseline: 2.2547x; 1.0105x over previous
import functools

import numpy as np
import jax
import jax.numpy as jnp
from jax import lax
from jax.experimental import pallas as pl
from jax.experimental.pallas import tpu as pltpu
from jax.experimental.pallas import tpu_sc as plsc

VOCAB = 1_000_000
DIM = 64
BATCH = 16384
NEG = 20
NCTX = NEG + 1
NW = 32
BPW = BATCH // NW
CH = 16
NCHUNK = BPW // CH
NIP = 8
SCORES = NCTX * CH
NSLICE = ((0, 128), (128, 128), (256, 64))

O_PERM = [0, 8, 4, 12, 2, 10, 6, 14, 1, 9, 5, 13, 3, 11, 7, 15]


TW = 8192
TGRID = -(-VOCAB // (2 * TW))
VPAD = TGRID * 2 * TW


def _tr_body(alo_ref, ahi_ref, blo_ref, bhi_ref, ao_ref, bo_ref):
  ao_ref[...] = jnp.concatenate([alo_ref[...].T, ahi_ref[...].T], axis=1)
  bo_ref[...] = jnp.concatenate([blo_ref[...].T, bhi_ref[...].T], axis=1)


def _transpose_tables(center_table, context_table):
  a = center_table.T
  b = context_table.T
  nin = -(-VOCAB // TW) - 1
  lo = pl.BlockSpec((DIM, TW), lambda w: (0, jnp.minimum(2 * w, nin)))
  hi = pl.BlockSpec((DIM, TW), lambda w: (0, jnp.minimum(2 * w + 1, nin)))
  out = pl.BlockSpec((TW, 2 * DIM), lambda w: (w, 0))
  return pl.pallas_call(
      _tr_body,
      grid=(TGRID,),
      in_specs=[lo, hi, lo, hi],
      out_specs=[out, out],
      out_shape=[jax.ShapeDtypeStruct((VPAD // 2, 2 * DIM), jnp.float32)] * 2,
  )(a, a, b, b)


def _sc_scores(center_words, context_words, neg2d, center_table, context_table):
  mesh = plsc.VectorSubcoreMesh(core_axis_name="c", subcore_axis_name="s")

  scratch = []
  for _ in range(2):
    scratch += [
        pltpu.VMEM((CH,), jnp.int32),
        pltpu.VMEM((CH,), jnp.int32),
        pltpu.VMEM((NIP, 128), jnp.int32),
        pltpu.VMEM((CH,), jnp.int32),
        pltpu.VMEM((CH,), jnp.int32),
        pltpu.VMEM((NIP, 128), jnp.int32),
        pltpu.VMEM((CH, 2 * DIM), jnp.float32),
        pltpu.VMEM((CH, 2 * DIM), jnp.float32),
        pltpu.VMEM((CH * NEG, 2 * DIM), jnp.float32),
        pltpu.VMEM((SCORES,), jnp.float32),
        pltpu.SemaphoreType.DMA,
    ]

  @functools.partial(
      pl.kernel,
      mesh=mesh,
      compiler_params=pltpu.CompilerParams(
          needs_layout_passes=False, use_tc_tiling_on_sc=False),
      out_type=jax.ShapeDtypeStruct((BATCH * NCTX,), jnp.float32),
      scratch_types=scratch,
  )
  def k(cw_hbm, xw_hbm, nw_hbm, ctab2_hbm, xtab2_hbm, out_hbm, *bufs):
    wid = lax.axis_index("s") * 2 + lax.axis_index("c")
    sets = (bufs[:11], bufs[11:])

    iota = lax.iota(jnp.int32, 16)
    rots, conds = {}, {}
    for g in (16, 8, 4, 2):
      rots[g] = (iota & ~(g - 1)) | ((iota + g // 2) & (g - 1))
      conds[g] = (iota & (g - 1)) < (g // 2)

    def _merge(a, b, g):
      asum = None if a is None else a + a.at[rots[g]].get(mode="promise_in_bounds")
      bsum = None if b is None else b + b.at[rots[g]].get(mode="promise_in_bounds")
      if asum is None and bsum is None:
        return None
      if bsum is None:
        return jnp.where(conds[g], asum, 0.0)
      if asum is None:
        return jnp.where(conds[g], 0.0, bsum)
      return jnp.where(conds[g], asum, bsum)

    def _tree(leaves):
      regs = list(leaves)
      g = 16
      while len(regs) > 1:
        regs = [_merge(regs[2 * i], regs[2 * i + 1], g)
                for i in range(len(regs) // 2)]
        g //= 2
      return regs[0]

    brev = (((iota & 1) * 8) | ((iota & 2) * 2)
            | ((iota & 4) // 2) | ((iota & 8) // 8))
    ovec1 = brev * CH
    mask2 = brev < 5
    ovec2 = jnp.where(mask2, (16 + brev) * CH, 0)

    def _nidx_slice(nu, off, cnt):
      if cnt == 128:
        return nu.at[off // 128]
      return nu.at[off // 128, pl.ds(0, cnt)]

    def fire(s, ci):
      cidx, xidx, nidx, cu, xu, nu, crows, xrows, nrows, _, sem = sets[s]
      base = pl.multiple_of(wid * BPW + ci * CH, CH)
      nbase = pl.multiple_of((wid * NCHUNK + ci) * NIP, NIP)
      pltpu.sync_copy(cw_hbm.at[pl.ds(base, CH)], cidx)
      pltpu.sync_copy(xw_hbm.at[pl.ds(base, CH)], xidx)
      pltpu.sync_copy(nw_hbm.at[pl.ds(nbase, NIP)], nidx)
      for q in range(CH // 16):
        cu[pl.ds(16 * q, 16)] = cidx[pl.ds(16 * q, 16)] // 2
        xu[pl.ds(16 * q, 16)] = xidx[pl.ds(16 * q, 16)] // 2
      for j in range(3):
        for q in range(8):
          nu[j, pl.ds(16 * q, 16)] = nidx[j, pl.ds(16 * q, 16)] // 2
      pltpu.async_copy(ctab2_hbm.at[cu], crows, sem)
      pltpu.async_copy(xtab2_hbm.at[xu], xrows, sem)
      for off, cnt in NSLICE:
        pltpu.async_copy(xtab2_hbm.at[_nidx_slice(nu, off, cnt)],
                         nrows.at[pl.ds(off, cnt)], sem)

    def drain(s):
      cidx, xidx, nidx, cu, xu, nu, crows, xrows, nrows, _, sem = sets[s]
      pltpu.make_async_copy(ctab2_hbm.at[cu], crows, sem).wait()
      pltpu.make_async_copy(xtab2_hbm.at[xu], xrows, sem).wait()
      for off, cnt in NSLICE:
        pltpu.make_async_copy(xtab2_hbm.at[_nidx_slice(nu, off, cnt)],
                              nrows.at[pl.ds(off, cnt)], sem).wait()

    def _pick(vec, lane):
      return vec.at[jnp.full((16,), lane, jnp.int32)].get(
          mode="promise_in_bounds")[0]

    def _half(ref, t, off):
      return [ref[t, pl.ds(off + 16 * q, 16)] for q in range(4)]

    def compute(s, ci):
      cidx, xidx, nidx, _, _, _, crows, xrows, nrows, scores, _ = sets[s]

      def row(r, carry):
        cb = (_pick(cidx[pl.ds(r // 16 * 16, 16)], r % 16) & 1) * 64
        xb = (_pick(xidx[pl.ds(r // 16 * 16, 16)], r % 16) & 1) * 64
        c = _half(crows, r, cb)
        x = _half(xrows, r, xb)
        ps = [c[0] * x[0] + c[1] * x[1] + c[2] * x[2] + c[3] * x[3]]
        for n in range(NEG):
          t = r * NEG + n
          nvec = nidx[t // 128, pl.ds((t % 128) // 16 * 16, 16)]
          nb = (_pick(nvec, t % 16) & 1) * 64
          v = _half(nrows, t, nb)
          ps.append(c[0] * v[0] + c[1] * v[1] + c[2] * v[2] + c[3] * v[3])
        f1 = _tree(ps[:16])
        f2 = _tree(ps[16:] + [None] * 11)
        plsc.store_scatter(scores, [ovec1 + r], f1)
        plsc.store_scatter(scores, [ovec2 + r], f2, mask=mask2)
        return carry

      lax.fori_loop(0, CH, row, 0)
      obase = pl.multiple_of((wid * NCHUNK + ci) * SCORES, 8)
      pltpu.sync_copy(scores, out_hbm.at[pl.ds(obase, SCORES)])

    fire(0, 0)

    def pair(ip, carry):
      ci = ip * 2
      fire(1, ci + 1)
      drain(0)
      compute(0, ci)

      @pl.when(ip < NCHUNK // 2 - 1)
      def _():
        fire(0, ci + 2)

      drain(1)
      compute(1, ci + 1)
      return carry

    lax.fori_loop(0, NCHUNK // 2, pair, 0)

  return k(center_words, context_words, neg2d, center_table, context_table)


def _loss_body(s_ref, o_ref):
  x = s_ref[...]
  flat = (lax.broadcasted_iota(jnp.int32, x.shape, 0) * 128
          + lax.broadcasted_iota(jnp.int32, x.shape, 1))
  n_id = (flat % SCORES) // CH
  t = jnp.where(n_id == 0, x, -x)
  p = 1.0 / (1.0 + jnp.exp(-t))
  ll = -jnp.log(p + 1e-10)
  o_ref[0, 0] = jnp.sum(ll) * (1.0 / BATCH)


def kernel(center_words, context_words, negative_samples, center_table, context_table):
  ct2, xt2 = _transpose_tables(center_table, context_table)

  def remap(v):
    v = v.astype(jnp.int32)
    p = v % (2 * TW)
    return 2 * TW * (v // (2 * TW)) + 2 * (p % TW) + p // TW

  cw = remap(center_words)
  xw = remap(context_words)
  ns = remap(negative_samples).reshape(BATCH // CH, CH * NEG)
  ns = jnp.pad(ns, ((0, 0), (0, NIP * 128 - CH * NEG)))
  ns = ns.reshape(BATCH // CH * NIP, 128)
  scores = _sc_scores(cw, xw, ns, ct2, xt2)
  s2 = scores.reshape(BATCH * NCTX // 128, 128)
  loss = pl.pallas_call(
      _loss_body,
      out_shape=jax.ShapeDtypeStruct((1, 1), jnp.float32),
      out_specs=pl.BlockSpec(memory_space=pltpu.SMEM),
  )(s2)
  return loss[0, 0]
